# baseline (speedup 1.0000x reference)
import functools

import jax
import jax.numpy as jnp
from jax.experimental import pallas as pl
from jax.experimental.pallas import tpu as pltpu

_KP1 = 4


def _fourth_smallest_sq(d2t):
    t = d2t
    m = None
    for it in range(_KP1):
        m = jnp.min(t, axis=0, keepdims=True)
        if it < _KP1 - 1:
            t = jnp.where(t <= m, jnp.inf, t)
    return m


_NRC, _NFC, _R2RC, _REC = 0, 1, 2, 3


def _body(real_ref, fake_ref, rr_ref, rf_ref, met_ref,
          col_ref, nrl_ref, nfl_ref, r2fl_ref, prec_ref,
          *, bm, nb, n):
    p = pl.program_id(0)
    i = pl.program_id(1)
    sl = pl.ds(i * bm, bm)

    @pl.when(p == 0)
    def _norms():
        rrow = real_ref[sl, :].astype(jnp.float32)
        ncol = jnp.sum(rrow * rrow, axis=1, keepdims=True)
        col_ref[sl, _NRC:_NRC + 1] = ncol
        nrl_ref[0, sl] = ncol[:, 0]
        frow = fake_ref[sl, :].astype(jnp.float32)
        ncol = jnp.sum(frow * frow, axis=1, keepdims=True)
        col_ref[sl, _NFC:_NFC + 1] = ncol
        nfl_ref[0, sl] = ncol[:, 0]

    def _d2t_strip(bank_ref, bank_ncol, rows_bf, rows_nl):
        g = jax.lax.dot_general(
            bank_ref[...], rows_bf,
            dimension_numbers=(((1,), (1,)), ((), ())),
            preferred_element_type=jnp.float32)
        return bank_ncol + rows_nl[None, :] - 2.0 * g

    def _radii_phase(src_ref, ncol_lane, nl_ref, radii_out_ref):
        d2t = _d2t_strip(src_ref, col_ref[:, ncol_lane:ncol_lane + 1],
                         src_ref[sl, :], nl_ref[0, sl])
        v4 = _fourth_smallest_sq(d2t)
        r2 = jnp.maximum(v4, 1e-12)
        radii_out_ref[0, sl] = jnp.sqrt(r2)[0, :]
        return r2

    @pl.when(p == 1)
    def _real_radii():
        r2 = _radii_phase(real_ref, _NRC, nrl_ref, rr_ref)
        col_ref[sl, _R2RC:_R2RC + 1] = r2.reshape(bm, 1)

    @pl.when(p == 2)
    def _fake_radii():
        r2 = _radii_phase(fake_ref, _NFC, nfl_ref, rf_ref)
        r2fl_ref[0, sl] = r2[0, :]

    @pl.when(p == 3)
    def _cross():
        d2t = _d2t_strip(real_ref, col_ref[:, _NRC:_NRC + 1],
                         fake_ref[sl, :], nfl_ref[0, sl])
        c2 = jnp.maximum(d2t, 1e-12)
        within_real = (c2 <= col_ref[:, _R2RC:_R2RC + 1]).astype(jnp.float32)
        prec_ref[0, sl] = jnp.max(within_real, axis=0)
        within_fake = (c2 <= r2fl_ref[0, sl][None, :]).astype(jnp.float32)
        rec_part = jnp.max(within_fake, axis=1, keepdims=True)

        @pl.when(i == 0)
        def _():
            col_ref[:, _REC:_REC + 1] = rec_part

        @pl.when(i > 0)
        def _():
            col_ref[:, _REC:_REC + 1] = jnp.maximum(
                col_ref[:, _REC:_REC + 1], rec_part)

        @pl.when(i == nb - 1)
        def _():
            precision = jnp.sum(prec_ref[...]) / n
            recall = jnp.sum(col_ref[:, _REC:_REC + 1]) / n
            lane = jax.lax.broadcasted_iota(jnp.int32, (1, 128), 1)
            met_ref[...] = jnp.where(
                lane == 0, precision, jnp.where(lane == 1, recall, 0.0))


def kernel(real_feats, fake_feats):
    n, d = real_feats.shape
    bm = 256 if n % 256 == 0 else n
    nb = n // bm

    real_bf = real_feats.astype(jnp.bfloat16)
    fake_bf = fake_feats.astype(jnp.bfloat16)

    body = functools.partial(_body, bm=bm, nb=nb, n=n)

    full = pl.BlockSpec((n, d), lambda p, i: (0, 0))
    vec = pl.BlockSpec((1, n), lambda p, i: (0, 0))
    met = pl.BlockSpec((1, 128), lambda p, i: (0, 0))

    rr, rf, metrics = pl.pallas_call(
        body,
        grid=(4, nb),
        in_specs=[full, full],
        out_specs=[vec, vec, met],
        out_shape=[
            jax.ShapeDtypeStruct((1, n), jnp.float32),
            jax.ShapeDtypeStruct((1, n), jnp.float32),
            jax.ShapeDtypeStruct((1, 128), jnp.float32),
        ],
        scratch_shapes=[
            pltpu.VMEM((n, 128), jnp.float32),
            pltpu.VMEM((1, n), jnp.float32),
            pltpu.VMEM((1, n), jnp.float32),
            pltpu.VMEM((1, n), jnp.float32),
            pltpu.VMEM((1, n), jnp.float32),
        ],
        compiler_params=pltpu.CompilerParams(
            dimension_semantics=("arbitrary", "arbitrary")),
    )(real_bf, fake_bf)

    return jnp.concatenate(
        [metrics[0, :2], rr[0, :], rf[0, :]])


# speedup vs baseline: 1.0420x; 1.0420x over previous
import functools

import jax
import jax.numpy as jnp
from jax.experimental import pallas as pl
from jax.experimental.pallas import tpu as pltpu

_KP1 = 4


def _fourth_smallest_sq(d2):
    t = d2
    m = None
    for it in range(_KP1):
        m = jnp.min(t, axis=1, keepdims=True)
        if it < _KP1 - 1:
            t = jnp.where(t <= m, jnp.inf, t)
    return m


def _body(real_ref, fake_ref, rr_ref, rf_ref, met_ref,
          nr_ref, nf_ref, r2r_ref, r2f_ref, prec_ref, rec_ref, g_ref,
          *, bm, nb, n):
    p = pl.program_id(0)
    i = pl.program_id(1)
    slot = jax.lax.rem(i, 2)
    prev_slot = 1 - slot
    sl = pl.ds(i * bm, bm)
    j = i - 1
    slj = pl.ds(j * bm, bm)

    @pl.when((p == 0) & (i < nb))
    def _norms():
        rrow = real_ref[sl, :].astype(jnp.float32)
        nr_ref[0, sl] = jnp.sum(rrow * rrow, axis=1)
        frow = fake_ref[sl, :].astype(jnp.float32)
        nf_ref[0, sl] = jnp.sum(frow * frow, axis=1)

    def _gram(rows_ref, cols_ref):
        g_ref[slot] = jax.lax.dot_general(
            rows_ref[sl, :], cols_ref[...],
            dimension_numbers=(((1,), (1,)), ((), ())),
            preferred_element_type=jnp.float32)

    def _d2_prev(rownorm_ref, colnorm_ref):
        xn = rownorm_ref[0, slj].reshape(bm, 1)
        return xn + colnorm_ref[...] - 2.0 * g_ref[prev_slot]

    def _radii_tail(norm_ref, radii_out_ref, r2_out_ref):
        d2 = _d2_prev(norm_ref, norm_ref)
        v4 = _fourth_smallest_sq(d2)
        r2 = jnp.maximum(v4, 1e-12)
        r2_out_ref[0, slj] = r2[:, 0]
        radii_out_ref[0, slj] = jnp.sqrt(r2)[:, 0]

    @pl.when(p == 1)
    def _real_radii():
        @pl.when(i < nb)
        def _():
            _gram(real_ref, real_ref)

        @pl.when(i > 0)
        def _():
            _radii_tail(nr_ref, rr_ref, r2r_ref)

    @pl.when(p == 2)
    def _fake_radii():
        @pl.when(i < nb)
        def _():
            _gram(fake_ref, fake_ref)

        @pl.when(i > 0)
        def _():
            _radii_tail(nf_ref, rf_ref, r2f_ref)

    @pl.when(p == 3)
    def _cross():
        @pl.when(i < nb)
        def _():
            _gram(fake_ref, real_ref)

        @pl.when(i > 0)
        def _():
            d2 = _d2_prev(nf_ref, nr_ref)
            c2 = jnp.maximum(d2, 1e-12)
            within_real = (c2 <= r2r_ref[...]).astype(jnp.float32)
            prec_ref[0, slj] = jnp.max(within_real, axis=1)
            r2f_block = r2f_ref[0, slj].reshape(bm, 1)
            within_fake = (c2 <= r2f_block).astype(jnp.float32)
            rec_part = jnp.max(within_fake, axis=0, keepdims=True)

            @pl.when(i == 1)
            def _():
                rec_ref[...] = rec_part

            @pl.when(i > 1)
            def _():
                rec_ref[...] = jnp.maximum(rec_ref[...], rec_part)

            @pl.when(i == nb)
            def _():
                precision = jnp.sum(prec_ref[...]) / n
                recall = jnp.sum(rec_ref[...]) / n
                lane = jax.lax.broadcasted_iota(jnp.int32, (1, 128), 1)
                met_ref[...] = jnp.where(
                    lane == 0, precision, jnp.where(lane == 1, recall, 0.0))


def kernel(real_feats, fake_feats):
    n, d = real_feats.shape
    bm = 256 if n % 256 == 0 else n
    nb = n // bm

    real_bf = real_feats.astype(jnp.bfloat16)
    fake_bf = fake_feats.astype(jnp.bfloat16)

    body = functools.partial(_body, bm=bm, nb=nb, n=n)

    full = pl.BlockSpec((n, d), lambda p, i: (0, 0))
    vec = pl.BlockSpec((1, n), lambda p, i: (0, 0))
    met = pl.BlockSpec((1, 128), lambda p, i: (0, 0))

    rr, rf, metrics = pl.pallas_call(
        body,
        grid=(4, nb + 1),
        in_specs=[full, full],
        out_specs=[vec, vec, met],
        out_shape=[
            jax.ShapeDtypeStruct((1, n), jnp.float32),
            jax.ShapeDtypeStruct((1, n), jnp.float32),
            jax.ShapeDtypeStruct((1, 128), jnp.float32),
        ],
        scratch_shapes=[
            pltpu.VMEM((1, n), jnp.float32),
            pltpu.VMEM((1, n), jnp.float32),
            pltpu.VMEM((1, n), jnp.float32),
            pltpu.VMEM((1, n), jnp.float32),
            pltpu.VMEM((1, n), jnp.float32),
            pltpu.VMEM((1, n), jnp.float32),
            pltpu.VMEM((2, bm, n), jnp.float32),
        ],
        compiler_params=pltpu.CompilerParams(
            dimension_semantics=("arbitrary", "arbitrary")),
    )(real_bf, fake_bf)

    return jnp.concatenate(
        [metrics[0, :2], rr[0, :], rf[0, :]])
